# Initial kernel scaffold; baseline (speedup 1.0000x reference)
import jax
import jax.numpy as jnp
from jax.experimental import pallas as pl


def kernel(x, pos, batch, W1, b1, W2, b2):
    raise NotImplementedError("write your pallas kernel here")


# speedup vs baseline: 3.1532x; 3.1532x over previous
import functools
import math

import jax
import jax.numpy as jnp
from jax.experimental import pallas as pl
from jax.experimental.pallas import tpu as pltpu

_N = 10000
_D = 128
_S = 5000
_K = 64
_H = 128
_R2 = 0.04
_SUB = 8
_LAN = 1280
_NP = _SUB * _LAN

_HIGH = jax.lax.Precision.HIGHEST


def _fps_body(posm_ref, poss_ref, idx_ref):
    lane_i = (jax.lax.broadcasted_iota(jnp.int32, (_SUB, _LAN), 0) * _LAN
              + jax.lax.broadcasted_iota(jnp.int32, (_SUB, _LAN), 1))
    px0 = poss_ref[0, 0]
    py0 = poss_ref[1, 0]
    pz0 = poss_ref[2, 0]
    big_px = posm_ref[0]
    big_py = posm_ref[1]
    big_pz = posm_ref[2]
    d0 = ((big_px - px0) ** 2 + (big_py - py0) ** 2 + (big_pz - pz0) ** 2)
    d0 = jnp.where(lane_i < _N, d0, -jnp.inf)
    idx_ref[0] = 0

    def body(i, d):
        m = jnp.max(d)
        nxt = jnp.min(jnp.where(d == m, lane_i, jnp.int32(2 ** 30)))
        idx_ref[i] = nxt
        px = poss_ref[0, nxt]
        py = poss_ref[1, nxt]
        pz = poss_ref[2, nxt]
        dn = (big_px - px) ** 2 + (big_py - py) ** 2 + (big_pz - pz) ** 2
        return jnp.minimum(d, dn)

    jax.lax.fori_loop(1, _S, body, d0)


def _fps(pos):
    post = jnp.transpose(pos)
    posm = jnp.pad(post, ((0, 0), (0, _NP - _N))).reshape(3, _SUB, _LAN)
    return pl.pallas_call(
        _fps_body,
        out_shape=jax.ShapeDtypeStruct((_S,), jnp.int32),
        in_specs=[
            pl.BlockSpec(memory_space=pltpu.VMEM),
            pl.BlockSpec(memory_space=pltpu.SMEM),
        ],
        out_specs=pl.BlockSpec(memory_space=pltpu.SMEM),
    )(posm, post)


def _table_body(x_ref, pos_ref, w1_ref, b1_ref, t_ref):
    wx = w1_ref[0:_D, :]
    wp = w1_ref[_D:_D + 3, :]
    t_ref[...] = (
        jnp.dot(x_ref[...], wx, preferred_element_type=jnp.float32,
                precision=_HIGH)
        + jnp.dot(pos_ref[...], wp, preferred_element_type=jnp.float32,
                  precision=_HIGH)
        + b1_ref[...])


def _table(x, pos, w1, b1):
    return pl.pallas_call(
        _table_body,
        out_shape=jax.ShapeDtypeStruct((_N, _H), jnp.float32),
        in_specs=[pl.BlockSpec(memory_space=pltpu.VMEM)] * 4,
        out_specs=pl.BlockSpec(memory_space=pltpu.VMEM),
    )(x, pos, w1, b1.reshape(1, _H))


_CB = 40
_GRID = _S // _CB


def _mlp_body(cen_ref, negd_ref, w1_ref, w2_ref, b2_ref, g_ref, out_ref):
    wp = w1_ref[_D:_D + 3, :]
    qc = jnp.dot(cen_ref[...], wp, preferred_element_type=jnp.float32,
                 precision=_HIGH)
    g = g_ref[...]
    h1 = jnp.maximum(g.reshape(_CB, _K, _H) - qc[:, None, :], 0.0)
    h2 = jnp.dot(h1.reshape(_CB * _K, _H), w2_ref[...],
                 preferred_element_type=jnp.float32, precision=_HIGH)
    h2 = jnp.maximum(h2 + b2_ref[...], 0.0)
    valid = negd_ref[...] >= -_R2
    h2 = jnp.where(valid, h2.reshape(_CB, _K, _H), -1e30)
    out_ref[...] = jnp.max(h2, axis=1)


def _mlp(centers, negd, w1, w2, b2, g):
    return pl.pallas_call(
        _mlp_body,
        grid=(_GRID,),
        out_shape=jax.ShapeDtypeStruct((_S, _H), jnp.float32),
        in_specs=[
            pl.BlockSpec((_CB, 3), lambda i: (i, 0)),
            pl.BlockSpec((_CB, _K, 1), lambda i: (i, 0, 0)),
            pl.BlockSpec((_D + 3, _H), lambda i: (0, 0)),
            pl.BlockSpec((_H, _H), lambda i: (0, 0)),
            pl.BlockSpec((1, _H), lambda i: (0, 0)),
            pl.BlockSpec((_CB * _K, _H), lambda i: (i, 0)),
        ],
        out_specs=pl.BlockSpec((_CB, _H), lambda i: (i, 0)),
    )(centers, negd, w1, w2, b2.reshape(1, _H), g)


def kernel(x, pos, batch, W1, b1, W2, b2):
    idx = _fps(pos)
    centers = pos[idx]
    d2 = (jnp.sum(centers ** 2, axis=1)[:, None]
          + jnp.sum(pos ** 2, axis=1)[None, :]
          - 2.0 * centers @ pos.T)
    negd, cols = jax.lax.top_k(-d2, _K)
    t = _table(x, pos, W1, b1)
    g = t[cols.reshape(-1)]
    out = _mlp(centers, negd[:, :, None], W1, W2, b2, g)
    return (out, centers, batch[idx])
